# baseline (speedup 1.0000x reference)
import functools

import jax
import jax.numpy as jnp
from jax import lax
from jax.experimental import pallas as pl
from jax.experimental.pallas import tpu as pltpu
from jax.experimental.pallas import tpu_sc as plsc

B = 16384
D = 64
NB = 1_000_000
NC = 2
NS = 16
NW = NC * NS
BPW = B // NW
L = 16
NSTR = D * (BPW // 128)

MLP_BLK = 2048


def _gather_body(tab1d_hbm, idx_hbm, out_hbm, idx_v, fidx, blk, sem):
    wid = lax.axis_index("s") * NC + lax.axis_index("c")
    base = wid * BPW
    pltpu.sync_copy(idx_hbm.at[pl.ds(base, BPW)], idx_v)

    def kbody(k, carry):
        def gbody(g, carry2):
            v = idx_v[pl.ds(g * L, L)]
            fidx[k * 4 + g // 8, pl.ds((g % 8) * L, L)] = v + k * NB
            return carry2
        lax.fori_loop(0, BPW // L, gbody, 0)
        return carry
    lax.fori_loop(0, D, kbody, 0)

    def sbody(r, carry):
        pltpu.async_copy(tab1d_hbm.at[fidx.at[r]],
                         blk.at[r // 4, pl.ds((r % 4) * 128, 128)], sem)
        return carry
    lax.fori_loop(0, NSTR, sbody, 0)

    def wbody(r, carry):
        pltpu.make_async_copy(tab1d_hbm.at[fidx.at[r]],
                              blk.at[r // 4, pl.ds((r % 4) * 128, 128)],
                              sem).wait()
        return carry
    lax.fori_loop(0, NSTR, wbody, 0)
    pltpu.sync_copy(blk, out_hbm.at[:, pl.ds(base, BPW)])


@functools.cache
def _gather_kernel():
    mesh = plsc.VectorSubcoreMesh(
        core_axis_name="c", subcore_axis_name="s",
        num_cores=NC, num_subcores=NS)
    return pl.kernel(
        _gather_body,
        out_type=jax.ShapeDtypeStruct((D, B), jnp.float32),
        mesh=mesh,
        compiler_params=pltpu.CompilerParams(use_tc_tiling_on_sc=False),
        scratch_types=[
            pltpu.VMEM((BPW,), jnp.int32),
            pltpu.VMEM((NSTR, 128), jnp.int32),
            pltpu.VMEM((D, BPW), jnp.float32),
            pltpu.SemaphoreType.DMA,
        ],
    )


def _mlp_body(x_ref, g_ref, bt_ref, w1_ref, b1_ref, w2_ref, b2_ref, o_ref):
    x = x_ref[...]
    mu = jnp.mean(x, axis=0, keepdims=True)
    xc = x - mu
    var = jnp.mean(xc * xc, axis=0, keepdims=True)
    xn = xc * lax.rsqrt(var + 1e-5) * g_ref[...] + bt_ref[...]
    cn = (((0,), (0,)), ((), ()))
    h = lax.dot_general(w1_ref[...], xn, cn,
                        preferred_element_type=jnp.float32) + b1_ref[...]
    h = h * jax.nn.sigmoid(h)
    o_ref[...] = lax.dot_general(
        w2_ref[...], h, cn, preferred_element_type=jnp.float32) + b2_ref[...]


def _mlp(x, gamma, beta, W1, b1, W2, b2):
    full = lambda i: (0, 0)
    return pl.pallas_call(
        _mlp_body,
        grid=(B // MLP_BLK,),
        in_specs=[
            pl.BlockSpec((D, MLP_BLK), lambda i: (0, i)),
            pl.BlockSpec((D, 1), full),
            pl.BlockSpec((D, 1), full),
            pl.BlockSpec((D, D), full),
            pl.BlockSpec((D, 1), full),
            pl.BlockSpec((D, D), full),
            pl.BlockSpec((D, 1), full),
        ],
        out_specs=pl.BlockSpec((D, MLP_BLK), lambda i: (0, i)),
        out_shape=jax.ShapeDtypeStruct((D, B), jnp.float32),
    )(x, gamma.reshape(D, 1), beta.reshape(D, 1), W1,
      b1.reshape(D, 1), W2, b2.reshape(D, 1))


def kernel(bands, band_emb, gamma, beta, W1, b1, W2, b2):
    idx = bands.astype(jnp.int32)
    tab1d = band_emb.T.reshape(-1)
    gathered_t = _gather_kernel()(tab1d, idx)
    out_t = _mlp(gathered_t, gamma, beta, W1, b1, W2, b2)
    return out_t.T


# speedup vs baseline: 1.0012x; 1.0012x over previous
import functools

import jax
import jax.numpy as jnp
from jax import lax
from jax.experimental import pallas as pl
from jax.experimental.pallas import tpu as pltpu
from jax.experimental.pallas import tpu_sc as plsc

B = 16384
D = 64
NB = 1_000_000
NC = 2
NS = 16
NW = NC * NS
BPW = B // NW
L = 16
NSTR = D * (BPW // 128)

MLP_BLK = 2048


def _gather_body(tab_hbm, idx_hbm, out_hbm, idx_v, blk, sem):
    wid = lax.axis_index("s") * NC + lax.axis_index("c")
    base = wid * BPW
    for c in range(BPW // 128):
        pltpu.sync_copy(
            idx_hbm.at[pl.ds(base + c * 128, 128)], idx_v.at[c])

    def sbody(r, carry):
        k = r // (BPW // 128)
        c = r % (BPW // 128)
        pltpu.async_copy(tab_hbm.at[k].at[idx_v.at[c]],
                         blk.at[k, pl.ds(c * 128, 128)], sem)
        return carry
    lax.fori_loop(0, NSTR, sbody, 0)

    def wbody(r, carry):
        k = r // (BPW // 128)
        c = r % (BPW // 128)
        pltpu.make_async_copy(tab_hbm.at[k].at[idx_v.at[c]],
                              blk.at[k, pl.ds(c * 128, 128)], sem).wait()
        return carry
    lax.fori_loop(0, NSTR, wbody, 0)
    pltpu.sync_copy(blk, out_hbm.at[:, pl.ds(base, BPW)])


@functools.cache
def _gather_kernel():
    mesh = plsc.VectorSubcoreMesh(
        core_axis_name="c", subcore_axis_name="s",
        num_cores=NC, num_subcores=NS)
    return pl.kernel(
        _gather_body,
        out_type=jax.ShapeDtypeStruct((D, B), jnp.float32),
        mesh=mesh,
        compiler_params=pltpu.CompilerParams(use_tc_tiling_on_sc=False),
        scratch_types=[
            pltpu.VMEM((BPW // 128, 128), jnp.int32),
            pltpu.VMEM((D, BPW), jnp.float32),
            pltpu.SemaphoreType.DMA,
        ],
    )


def _mlp_body(x_ref, g_ref, bt_ref, w1_ref, b1_ref, w2_ref, b2_ref, o_ref):
    x = x_ref[...]
    mu = jnp.mean(x, axis=0, keepdims=True)
    xc = x - mu
    var = jnp.mean(xc * xc, axis=0, keepdims=True)
    xn = xc * lax.rsqrt(var + 1e-5) * g_ref[...] + bt_ref[...]
    cn = (((0,), (0,)), ((), ()))
    h = lax.dot_general(w1_ref[...], xn, cn,
                        preferred_element_type=jnp.float32) + b1_ref[...]
    h = h * jax.nn.sigmoid(h)
    o_ref[...] = lax.dot_general(
        w2_ref[...], h, cn, preferred_element_type=jnp.float32) + b2_ref[...]


def _mlp(x, gamma, beta, W1, b1, W2, b2):
    full = lambda i: (0, 0)
    return pl.pallas_call(
        _mlp_body,
        grid=(B // MLP_BLK,),
        in_specs=[
            pl.BlockSpec((D, MLP_BLK), lambda i: (0, i)),
            pl.BlockSpec((D, 1), full),
            pl.BlockSpec((D, 1), full),
            pl.BlockSpec((D, D), full),
            pl.BlockSpec((D, 1), full),
            pl.BlockSpec((D, D), full),
            pl.BlockSpec((D, 1), full),
        ],
        out_specs=pl.BlockSpec((D, MLP_BLK), lambda i: (0, i)),
        out_shape=jax.ShapeDtypeStruct((D, B), jnp.float32),
    )(x, gamma.reshape(D, 1), beta.reshape(D, 1), W1,
      b1.reshape(D, 1), W2, b2.reshape(D, 1))


def kernel(bands, band_emb, gamma, beta, W1, b1, W2, b2):
    idx = bands.astype(jnp.int32)
    gathered_t = _gather_kernel()(band_emb.T, idx)
    out_t = _mlp(gathered_t, gamma, beta, W1, b1, W2, b2)
    return out_t.T


# speedup vs baseline: 8.2849x; 8.2749x over previous
import functools

import jax
import jax.numpy as jnp
from jax import lax
from jax.experimental import pallas as pl
from jax.experimental.pallas import tpu as pltpu
from jax.experimental.pallas import tpu_sc as plsc

B = 16384
D = 64
NC = 2
NS = 16
NW = NC * NS
BPW = B // NW
L = 16

MLP_BLK = 2048


def _gather_body(table_hbm, idx_hbm, out_hbm, idx_v, sem):
    wid = lax.axis_index("s") * NC + lax.axis_index("c")
    base = wid * BPW
    pltpu.sync_copy(idx_hbm.at[pl.ds(base, BPW)], idx_v)

    def chunk_body(c, carry):
        v = idx_v[pl.ds(c * L, L)]
        copies = []
        for l in range(L):
            i = v[l]
            copies.append(pltpu.async_copy(
                table_hbm.at[pl.ds(i, 1)],
                out_hbm.at[pl.ds(base + c * L + l, 1)], sem))
        for cp in copies:
            cp.wait()
        return carry

    lax.fori_loop(0, BPW // L, chunk_body, 0)


@functools.cache
def _gather_kernel():
    mesh = plsc.VectorSubcoreMesh(
        core_axis_name="c", subcore_axis_name="s",
        num_cores=NC, num_subcores=NS)
    return pl.kernel(
        _gather_body,
        out_type=jax.ShapeDtypeStruct((B, D), jnp.float32),
        mesh=mesh,
        compiler_params=pltpu.CompilerParams(use_tc_tiling_on_sc=True),
        scratch_types=[
            pltpu.VMEM((BPW,), jnp.int32),
            pltpu.SemaphoreType.DMA,
        ],
    )


def _mlp_body(x_ref, g_ref, bt_ref, w1_ref, b1_ref, w2_ref, b2_ref, o_ref):
    x = x_ref[...]
    mu = jnp.mean(x, axis=-1, keepdims=True)
    xc = x - mu
    var = jnp.mean(xc * xc, axis=-1, keepdims=True)
    xn = xc * lax.rsqrt(var + 1e-5) * g_ref[...] + bt_ref[...]
    h = jnp.dot(xn, w1_ref[...], preferred_element_type=jnp.float32) + b1_ref[...]
    h = h * jax.nn.sigmoid(h)
    o_ref[...] = lax.dot_general(
        w2_ref[...], h, (((0,), (1,)), ((), ())),
        preferred_element_type=jnp.float32) + b2_ref[...]


def _mlp(x, gamma, beta, W1, b1, W2, b2):
    full = lambda i: (0, 0)
    return pl.pallas_call(
        _mlp_body,
        grid=(B // MLP_BLK,),
        in_specs=[
            pl.BlockSpec((MLP_BLK, D), lambda i: (i, 0)),
            pl.BlockSpec((1, D), full),
            pl.BlockSpec((1, D), full),
            pl.BlockSpec((D, D), full),
            pl.BlockSpec((1, D), full),
            pl.BlockSpec((D, D), full),
            pl.BlockSpec((D, 1), full),
        ],
        out_specs=pl.BlockSpec((D, MLP_BLK), lambda i: (0, i)),
        out_shape=jax.ShapeDtypeStruct((D, B), jnp.float32),
    )(x, gamma.reshape(1, D), beta.reshape(1, D), W1,
      b1.reshape(1, D), W2, b2.reshape(D, 1))


def kernel(bands, band_emb, gamma, beta, W1, b1, W2, b2):
    idx = bands.astype(jnp.int32)
    gathered = _gather_kernel()(band_emb, idx)
    out_t = _mlp(gathered, gamma, beta, W1, b1, W2, b2)
    return out_t.T
